# baseline (speedup 1.0000x reference)
import jax
import jax.numpy as jnp
from jax.experimental import pallas as pl
from jax.experimental.pallas import tpu as pltpu

PI_C = 3.141592653
NAG, NPER_C = 4, 256
NB = NAG * NPER_C
THR = 0.1
TILE = 256
CHK = 128

_INTERPRET = False

_SGN = ((0.5, 0.5), (-0.5, 0.5), (-0.5, -0.5), (0.5, -0.5))


def _limit_period(v):
    return v - jnp.floor(v / (2.0 * PI_C) + 0.5) * (2.0 * PI_C)


def _agent_shift(t_ref, sel_mask):
    tx = t_ref[:, 0:1]
    ty = t_ref[:, 1:2]
    a4 = jax.lax.broadcasted_iota(jnp.int32, (4, 1), 0)
    m0 = a4 == 0
    sx = (jnp.sum(jnp.where(sel_mask, tx, 0.0), keepdims=True)
          - jnp.sum(jnp.where(m0, tx, 0.0), keepdims=True))
    sy = (jnp.sum(jnp.where(sel_mask, ty, 0.0), keepdims=True)
          - jnp.sum(jnp.where(m0, ty, 0.0), keepdims=True))
    return sx.reshape(1, 1), sy.reshape(1, 1)


def _clip_contrib(px, py, qx, qy, ux, uy, betas, acc):
    dx, dy = qx - px, qy - py
    t_lo = None
    t_hi = None
    infea = None
    for k in range(4):
        k2 = (k + 1) % 4
        ex = ux[k2] - ux[k]
        ey = uy[k2] - uy[k]
        beta = betas[k]
        alpha = ex * (py - uy[k]) - ey * (px - ux[k])
        r = -alpha / jnp.where(beta == 0.0, 1.0, beta)
        lo_k = jnp.where(beta > 0.0, r, -1e9)
        hi_k = jnp.where(beta < 0.0, r, 1e9)
        bad_k = (beta == 0.0) & (alpha < 0.0)
        if t_lo is None:
            t_lo, t_hi, infea = lo_k, hi_k, bad_k
        else:
            t_lo = jnp.maximum(t_lo, lo_k)
            t_hi = jnp.minimum(t_hi, hi_k)
            infea = infea | bad_k
    t_lo = jnp.maximum(t_lo, 0.0)
    t_hi = jnp.minimum(t_hi, 1.0)
    t_lo = jnp.where(infea, 1e9, t_lo)
    return acc + jnp.where(t_hi > t_lo, (t_hi - t_lo) * (px * dy - py * dx), 0.0)


def _corners(x, y, dx, dy, h):
    c, s = jnp.cos(h), jnp.sin(h)
    cxs, cys = [], []
    for sx, sy in _SGN:
        lx = sx * dx
        ly = sy * dy
        cxs.append(c * lx - s * ly + x)
        cys.append(s * lx + c * ly + y)
    return cxs, cys


def _adj_kernel(br_ref, bcT_ref, t_ref, adj_ref):
    i = pl.program_id(0)
    j = pl.program_id(1)

    @pl.when(i > j)
    def _zero():
        adj_ref[:, :] = jnp.zeros((TILE, TILE), jnp.float32)

    @pl.when(i <= j)
    def _compute():
        _adj_tile(br_ref, bcT_ref, t_ref, adj_ref, i, j)


def _adj_tile(br_ref, bcT_ref, t_ref, adj_ref, i, j):
    a4 = jax.lax.broadcasted_iota(jnp.int32, (4, 1), 0)
    shxr, shyr = _agent_shift(t_ref, a4 == i)
    shxc, shyc = _agent_shift(t_ref, a4 == j)

    br = br_ref[:, :]
    xr = br[:, 0:1] + shxr
    yr = br[:, 1:2] + shyr
    zr, dxr, dyr, dzr, hr = (br[:, 2:3], br[:, 3:4], br[:, 4:5], br[:, 5:6],
                             br[:, 6:7])
    bc = bcT_ref[:, :]
    xc = bc[0:1, :] + shxc
    yc = bc[1:2, :] + shyc
    zc, dxc, dyc, dzc, hc = (bc[2:3, :], bc[3:4, :], bc[4:5, :], bc[5:6, :],
                             bc[6:7, :])

    arx, ary = _corners(xr, yr, dxr, dyr, hr)
    bcx, bcy = _corners(xc, yc, dxc, dyc, hc)

    mx = 0.5 * (xr + xc)
    my = 0.5 * (yr + yc)
    acx = [a - mx for a in arx]
    acy = [a - my for a in ary]
    ccx = [b - mx for b in bcx]
    ccy = [b - my for b in bcy]

    erx = [arx[(e + 1) % 4] - arx[e] for e in range(4)]
    ery = [ary[(e + 1) % 4] - ary[e] for e in range(4)]
    ecx = [bcx[(k + 1) % 4] - bcx[k] for k in range(4)]
    ecy = [bcy[(k + 1) % 4] - bcy[k] for k in range(4)]
    cr = [[ecx[k] * ery[e] - ecy[k] * erx[e] for e in range(4)]
          for k in range(4)]

    acc1 = jnp.zeros_like(mx)
    acc2 = jnp.zeros_like(mx)
    for e in range(4):
        e2 = (e + 1) % 4
        acc1 = _clip_contrib(acx[e], acy[e], acx[e2], acy[e2],
                             tuple(ccx), tuple(ccy),
                             [cr[k][e] for k in range(4)], acc1)
        acc2 = _clip_contrib(ccx[e], ccy[e], ccx[e2], ccy[e2],
                             tuple(acx), tuple(acy),
                             [-cr[e][k] for k in range(4)], acc2)
    inter = 0.5 * (acc1 + acc2)

    ih = jnp.maximum(jnp.minimum(zr + dzr * 0.5, zc + dzc * 0.5)
                     - jnp.maximum(zr - dzr * 0.5, zc - dzc * 0.5), 0.0)
    iv = inter * ih
    va = dxr * dyr * dzr
    vb = dxc * dyc * dzc
    iou = iv / jnp.maximum(va + vb - iv, 1e-6)
    adj_ref[:, :] = (iou > THR).astype(jnp.float32)


def _fuse_kernel(adj_ref, b_ref, bT_ref, sc_ref, sr_ref, t_ref, out_ref,
                 adjs_ref):
    f32 = jnp.float32
    lane_n = jax.lax.broadcasted_iota(jnp.int32, (1, NB), 1)

    for s0 in range(0, NB, CHK):
        rows = adj_ref[s0:s0 + CHK, :]
        cols = adj_ref[:, s0:s0 + CHK]
        adjs_ref[s0:s0 + CHK, :] = jnp.maximum(rows, cols.T)

    covered = jnp.zeros((1, NB), f32)
    lead_chunks = []
    for c0 in range(0, NB, CHK):
        covW = jax.lax.slice(covered, (0, c0), (1, c0 + CHK))
        leads = []
        for k in range(CHK):
            row = adjs_ref[c0 + k:c0 + k + 1, c0:c0 + CHK]
            rowr = pltpu.roll(row, (CHK - k) % CHK, axis=1)
            islead = 1.0 - covW[:, 0:1]
            covW = jnp.maximum(covW, islead * rowr)
            leads.append(islead)
            covW = pltpu.roll(covW, CHK - 1, axis=1)
        leadr = jnp.concatenate(leads, axis=1)
        lead_chunks.append(leadr)
        if c0 + CHK < NB:
            rows = adjs_ref[pl.ds(c0, CHK), :]
            hits = jnp.dot(leadr, rows, preferred_element_type=f32)
            covered = jnp.maximum(covered, (hits > 0.0).astype(f32))
    leader_row = jnp.concatenate(lead_chunks, axis=1)

    s_row = sr_ref[:, :]

    def colT(row_vec, s0):
        return jax.lax.slice(row_vec, (0, s0), (1, s0 + CHK)).T

    def sub_iota(s0):
        return jax.lax.broadcasted_iota(jnp.int32, (CHK, 1), 0) + s0

    cum_row = jnp.zeros((1, NB), f32)
    for s0 in range(0, NB, CHK):
        cum_row = cum_row + jnp.sum(
            jnp.where(sub_iota(s0) <= lane_n, colT(leader_row, s0), 0.0),
            axis=0, keepdims=True)
    val_row = leader_row * cum_row

    segf_row = jnp.zeros((1, NB), f32)
    for s0 in range(0, NB, CHK):
        sl = adjs_ref[s0:s0 + CHK, :]
        segf_row = jnp.maximum(
            segf_row,
            jnp.max(jnp.where(sl > 0.0, colT(val_row, s0), 0.0),
                    axis=0, keepdims=True))
    segf_row = segf_row - 1.0

    smax_row = jnp.zeros((1, NB), f32)
    rank_row = jnp.zeros((1, NB), f32)
    for s0 in range(0, NB, CHK):
        same_s = colT(segf_row, s0) == segf_row
        s_c = sc_ref[s0:s0 + CHK, :]
        smax_row = jnp.maximum(
            smax_row,
            jnp.max(jnp.where(same_s, s_c, 0.0), axis=0, keepdims=True))
        gt_s = (s_c > s_row) | ((s_c == s_row) & (sub_iota(s0) < lane_n))
        rank_row = rank_row + jnp.sum((same_s & gt_s).astype(f32),
                                      axis=0, keepdims=True)

    amin_row = jnp.zeros((1, NB), f32) + float(NB)
    for s0 in range(0, NB, CHK):
        same_s = colT(segf_row, s0) == segf_row
        s_c = sc_ref[s0:s0 + CHK, :]
        selc = same_s & (s_c == smax_row)
        amin_row = jnp.minimum(
            amin_row,
            jnp.min(jnp.where(selc, sub_iota(s0).astype(f32), float(NB)),
                    axis=0, keepdims=True))

    ref_row = jnp.zeros((1, NB), f32)
    for s0 in range(0, NB, CHK):
        same_s = colT(segf_row, s0) == segf_row
        s_c = sc_ref[s0:s0 + CHK, :]
        selc = same_s & (s_c == smax_row)
        dirs_c = b_ref[s0:s0 + CHK, 6:7]
        ref_row = ref_row + jnp.sum(
            jnp.where(selc & (sub_iota(s0).astype(f32) == amin_row),
                      dirs_c, 0.0), axis=0, keepdims=True)

    sg_row = jnp.zeros((1, NB), f32)
    sle_row = jnp.zeros((1, NB), f32)
    ssum_row = jnp.zeros((1, NB), f32)
    for s0 in range(0, NB, CHK):
        same_s = colT(segf_row, s0) == segf_row
        s_c = sc_ref[s0:s0 + CHK, :]
        dirs_c = b_ref[s0:s0 + CHK, 6:7]
        dd = jnp.abs(dirs_c - colT(ref_row, s0))
        dd = jnp.where(dd > PI_C, 2.0 * PI_C - dd, dd)
        mgt_c = (dd > PI_C / 2.0).astype(f32)
        w = jnp.where(same_s, s_c, 0.0)
        sg_row = sg_row + jnp.sum(w * mgt_c, axis=0, keepdims=True)
        sle_row = sle_row + jnp.sum(w * (1.0 - mgt_c), axis=0, keepdims=True)
        ssum_row = ssum_row + jnp.sum(w, axis=0, keepdims=True)

    a4 = jax.lax.broadcasted_iota(jnp.int32, (4, 1), 0)
    sub_nf = jax.lax.broadcasted_iota(jnp.int32, (NB, 1), 0).astype(f32)
    out9 = jnp.zeros((NB, 9), f32)
    for s0 in range(0, NB, CHK):
        seg_slice = jax.lax.slice(segf_row, (0, s0), (1, s0 + CHK))
        mm_s = (sub_nf == seg_slice).astype(f32)
        s_c = sc_ref[s0:s0 + CHK, :]
        dirs_c = b_ref[s0:s0 + CHK, 6:7]
        dd = jnp.abs(dirs_c - colT(ref_row, s0))
        dd = jnp.where(dd > PI_C, 2.0 * PI_C - dd, dd)
        mgt_c = (dd > PI_C / 2.0).astype(f32)
        addf = jnp.where(colT(sg_row, s0) <= colT(sle_row, s0),
                         mgt_c, 1.0 - mgt_c)
        dirs2 = _limit_period(dirs_c + addf * PI_C)
        ssum_c = colT(ssum_row, s0)
        snorm = s_c / jnp.where(ssum_c > 0.0, ssum_c, 1.0)
        term = jnp.exp((colT(rank_row, s0) + 1.0) * jnp.log(s_c))
        sx_a, sy_a = _agent_shift(t_ref, a4 == (s0 // NPER_C))
        x2_s = jnp.concatenate(
            [(b_ref[s0:s0 + CHK, 0:1] + sx_a) * snorm,
             (b_ref[s0:s0 + CHK, 1:2] + sy_a) * snorm,
             b_ref[s0:s0 + CHK, 2:3] * snorm,
             b_ref[s0:s0 + CHK, 3:4] * snorm,
             b_ref[s0:s0 + CHK, 4:5] * snorm,
             b_ref[s0:s0 + CHK, 5:6] * snorm,
             jnp.sin(dirs2) * snorm, jnp.cos(dirs2) * snorm, term], axis=1)
        out9 = out9 + jnp.dot(mm_s, x2_s, preferred_element_type=f32)

    theta = jnp.arctan2(out9[:, 6:7], out9[:, 7:8])
    sf = jnp.minimum(out9[:, 8:9], 1.0)
    out_ref[:, :] = jnp.concatenate([out9[:, 0:6], theta, sf], axis=1)


def kernel(det_boxes, det_scores, translations):
    f32 = jnp.float32
    boxes = det_boxes.astype(f32).reshape(NB, 7)
    bT = boxes.T
    s = det_scores.astype(f32).reshape(NB)
    s_col = s.reshape(NB, 1)
    s_row = s.reshape(1, NB)
    t = translations.astype(f32)

    g = NB // TILE
    adj = pl.pallas_call(
        _adj_kernel,
        grid=(g, g),
        in_specs=[
            pl.BlockSpec((TILE, 7), lambda i, j: (i, 0)),
            pl.BlockSpec((7, TILE), lambda i, j: (0, j)),
            pl.BlockSpec((4, 3), lambda i, j: (0, 0)),
        ],
        out_specs=pl.BlockSpec((TILE, TILE), lambda i, j: (i, j)),
        out_shape=jax.ShapeDtypeStruct((NB, NB), f32),
        compiler_params=pltpu.CompilerParams(
            dimension_semantics=("parallel", "arbitrary")),
        interpret=_INTERPRET,
    )(boxes, bT, t)

    out = pl.pallas_call(
        _fuse_kernel,
        out_shape=jax.ShapeDtypeStruct((NB, 8), f32),
        scratch_shapes=[pltpu.VMEM((NB, NB), f32)],
        interpret=_INTERPRET,
    )(adj, boxes, bT, s_col, s_row, t)
    return out


# speedup vs baseline: 1.3670x; 1.3670x over previous
import jax
import jax.numpy as jnp
from jax.experimental import pallas as pl
from jax.experimental.pallas import tpu as pltpu

PI_C = 3.141592653
NAG, NPER_C = 4, 256
NB = NAG * NPER_C
THR = 0.1
TILE = 256
CHK = 128

_INTERPRET = False

_SGN = ((0.5, 0.5), (-0.5, 0.5), (-0.5, -0.5), (0.5, -0.5))


def _limit_period(v):
    return v - jnp.floor(v / (2.0 * PI_C) + 0.5) * (2.0 * PI_C)


def _agent_shift(t_ref, sel_mask):
    tx = t_ref[:, 0:1]
    ty = t_ref[:, 1:2]
    a4 = jax.lax.broadcasted_iota(jnp.int32, (4, 1), 0)
    m0 = a4 == 0
    sx = (jnp.sum(jnp.where(sel_mask, tx, 0.0), keepdims=True)
          - jnp.sum(jnp.where(m0, tx, 0.0), keepdims=True))
    sy = (jnp.sum(jnp.where(sel_mask, ty, 0.0), keepdims=True)
          - jnp.sum(jnp.where(m0, ty, 0.0), keepdims=True))
    return sx.reshape(1, 1), sy.reshape(1, 1)


def _clip_contrib(px, py, qx, qy, ux, uy, betas, acc):
    dx, dy = qx - px, qy - py
    t_lo = None
    t_hi = None
    infea = None
    for k in range(4):
        k2 = (k + 1) % 4
        ex = ux[k2] - ux[k]
        ey = uy[k2] - uy[k]
        beta = betas[k]
        alpha = ex * (py - uy[k]) - ey * (px - ux[k])
        r = -alpha / jnp.where(beta == 0.0, 1.0, beta)
        lo_k = jnp.where(beta > 0.0, r, -1e9)
        hi_k = jnp.where(beta < 0.0, r, 1e9)
        bad_k = (beta == 0.0) & (alpha < 0.0)
        if t_lo is None:
            t_lo, t_hi, infea = lo_k, hi_k, bad_k
        else:
            t_lo = jnp.maximum(t_lo, lo_k)
            t_hi = jnp.minimum(t_hi, hi_k)
            infea = infea | bad_k
    t_lo = jnp.maximum(t_lo, 0.0)
    t_hi = jnp.minimum(t_hi, 1.0)
    t_lo = jnp.where(infea, 1e9, t_lo)
    return acc + jnp.where(t_hi > t_lo, (t_hi - t_lo) * (px * dy - py * dx), 0.0)


def _corners(x, y, dx, dy, h):
    c, s = jnp.cos(h), jnp.sin(h)
    cxs, cys = [], []
    for sx, sy in _SGN:
        lx = sx * dx
        ly = sy * dy
        cxs.append(c * lx - s * ly + x)
        cys.append(s * lx + c * ly + y)
    return cxs, cys


def _adj_kernel(br_ref, bcT_ref, t_ref, adj_ref):
    i = pl.program_id(0)
    j = pl.program_id(1)

    @pl.when(i > j)
    def _zero():
        adj_ref[:, :] = jnp.zeros((TILE, TILE), jnp.float32)

    @pl.when(i <= j)
    def _compute():
        _adj_tile(br_ref, bcT_ref, t_ref, adj_ref, i, j)


def _adj_tile(br_ref, bcT_ref, t_ref, adj_ref, i, j):
    a4 = jax.lax.broadcasted_iota(jnp.int32, (4, 1), 0)
    shxr, shyr = _agent_shift(t_ref, a4 == i)
    shxc, shyc = _agent_shift(t_ref, a4 == j)

    br = br_ref[:, :]
    xr = br[:, 0:1] + shxr
    yr = br[:, 1:2] + shyr
    zr, dxr, dyr, dzr, hr = (br[:, 2:3], br[:, 3:4], br[:, 4:5], br[:, 5:6],
                             br[:, 6:7])
    bc = bcT_ref[:, :]
    xc = bc[0:1, :] + shxc
    yc = bc[1:2, :] + shyc
    zc, dxc, dyc, dzc, hc = (bc[2:3, :], bc[3:4, :], bc[4:5, :], bc[5:6, :],
                             bc[6:7, :])

    arx, ary = _corners(xr, yr, dxr, dyr, hr)
    bcx, bcy = _corners(xc, yc, dxc, dyc, hc)

    mx = 0.5 * (xr + xc)
    my = 0.5 * (yr + yc)
    acx = [a - mx for a in arx]
    acy = [a - my for a in ary]
    ccx = [b - mx for b in bcx]
    ccy = [b - my for b in bcy]

    erx = [arx[(e + 1) % 4] - arx[e] for e in range(4)]
    ery = [ary[(e + 1) % 4] - ary[e] for e in range(4)]
    ecx = [bcx[(k + 1) % 4] - bcx[k] for k in range(4)]
    ecy = [bcy[(k + 1) % 4] - bcy[k] for k in range(4)]
    cr = [[ecx[k] * ery[e] - ecy[k] * erx[e] for e in range(4)]
          for k in range(4)]

    acc1 = jnp.zeros_like(mx)
    acc2 = jnp.zeros_like(mx)
    for e in range(4):
        e2 = (e + 1) % 4
        acc1 = _clip_contrib(acx[e], acy[e], acx[e2], acy[e2],
                             tuple(ccx), tuple(ccy),
                             [cr[k][e] for k in range(4)], acc1)
        acc2 = _clip_contrib(ccx[e], ccy[e], ccx[e2], ccy[e2],
                             tuple(acx), tuple(acy),
                             [-cr[e][k] for k in range(4)], acc2)
    inter = 0.5 * (acc1 + acc2)

    ih = jnp.maximum(jnp.minimum(zr + dzr * 0.5, zc + dzc * 0.5)
                     - jnp.maximum(zr - dzr * 0.5, zc - dzc * 0.5), 0.0)
    iv = inter * ih
    va = dxr * dyr * dzr
    vb = dxc * dyc * dzc
    iou = iv / jnp.maximum(va + vb - iv, 1e-6)
    adj_ref[:, :] = (iou > THR).astype(jnp.float32)


def _fuse_kernel(adj_ref, b_ref, bT_ref, sc_ref, sr_ref, t_ref, out_ref,
                 adjs_ref):
    f32 = jnp.float32
    lane_n = jax.lax.broadcasted_iota(jnp.int32, (1, NB), 1)

    for s0 in range(0, NB, CHK):
        rows = adj_ref[s0:s0 + CHK, :]
        cols = adj_ref[:, s0:s0 + CHK]
        adjs_ref[s0:s0 + CHK, :] = jnp.maximum(rows, cols.T)

    covered = jnp.zeros((1, NB), f32)
    lead_chunks = []
    for c0 in range(0, NB, CHK):
        covc = jax.lax.slice(covered, (0, c0), (1, c0 + CHK))
        lead_groups = []
        for g in range(CHK // 8):
            g8 = g * 8
            blk = adjs_ref[c0 + g8:c0 + g8 + 8, c0:c0 + CHK]
            b8 = blk[:, g8:g8 + 8]
            ls = []
            for r in range(8):
                p = 1.0 - jax.lax.slice(covc, (0, g8 + r), (1, g8 + r + 1))
                for s in range(r):
                    bsr = jax.lax.slice(b8, (s, r), (s + 1, r + 1))
                    p = p * (1.0 - ls[s] * bsr)
                ls.append(p)
            upd = ls[0] * blk[0:1, :]
            for r in range(1, 8):
                upd = jnp.maximum(upd, ls[r] * blk[r:r + 1, :])
            covc = jnp.maximum(covc, upd)
            lead_groups.append(jnp.concatenate(ls, axis=1))
        leadr = jnp.concatenate(lead_groups, axis=1)
        lead_chunks.append(leadr)
        if c0 + CHK < NB:
            rows = adjs_ref[pl.ds(c0, CHK), :]
            hits = jnp.dot(leadr, rows, preferred_element_type=f32)
            covered = jnp.maximum(covered, (hits > 0.0).astype(f32))
    leader_row = jnp.concatenate(lead_chunks, axis=1)

    s_row = sr_ref[:, :]

    def colT(row_vec, s0):
        return jax.lax.slice(row_vec, (0, s0), (1, s0 + CHK)).T

    def sub_iota(s0):
        return jax.lax.broadcasted_iota(jnp.int32, (CHK, 1), 0) + s0

    cum_row = jnp.zeros((1, NB), f32)
    for s0 in range(0, NB, CHK):
        cum_row = cum_row + jnp.sum(
            jnp.where(sub_iota(s0) <= lane_n, colT(leader_row, s0), 0.0),
            axis=0, keepdims=True)
    val_row = leader_row * cum_row

    segf_row = jnp.zeros((1, NB), f32)
    for s0 in range(0, NB, CHK):
        sl = adjs_ref[s0:s0 + CHK, :]
        segf_row = jnp.maximum(
            segf_row,
            jnp.max(jnp.where(sl > 0.0, colT(val_row, s0), 0.0),
                    axis=0, keepdims=True))
    segf_row = segf_row - 1.0

    smax_row = jnp.zeros((1, NB), f32)
    rank_row = jnp.zeros((1, NB), f32)
    for s0 in range(0, NB, CHK):
        same_s = colT(segf_row, s0) == segf_row
        s_c = sc_ref[s0:s0 + CHK, :]
        smax_row = jnp.maximum(
            smax_row,
            jnp.max(jnp.where(same_s, s_c, 0.0), axis=0, keepdims=True))
        gt_s = (s_c > s_row) | ((s_c == s_row) & (sub_iota(s0) < lane_n))
        rank_row = rank_row + jnp.sum((same_s & gt_s).astype(f32),
                                      axis=0, keepdims=True)

    amin_row = jnp.zeros((1, NB), f32) + float(NB)
    for s0 in range(0, NB, CHK):
        same_s = colT(segf_row, s0) == segf_row
        s_c = sc_ref[s0:s0 + CHK, :]
        selc = same_s & (s_c == smax_row)
        amin_row = jnp.minimum(
            amin_row,
            jnp.min(jnp.where(selc, sub_iota(s0).astype(f32), float(NB)),
                    axis=0, keepdims=True))

    ref_row = jnp.zeros((1, NB), f32)
    for s0 in range(0, NB, CHK):
        same_s = colT(segf_row, s0) == segf_row
        s_c = sc_ref[s0:s0 + CHK, :]
        selc = same_s & (s_c == smax_row)
        dirs_c = b_ref[s0:s0 + CHK, 6:7]
        ref_row = ref_row + jnp.sum(
            jnp.where(selc & (sub_iota(s0).astype(f32) == amin_row),
                      dirs_c, 0.0), axis=0, keepdims=True)

    sg_row = jnp.zeros((1, NB), f32)
    sle_row = jnp.zeros((1, NB), f32)
    ssum_row = jnp.zeros((1, NB), f32)
    for s0 in range(0, NB, CHK):
        same_s = colT(segf_row, s0) == segf_row
        s_c = sc_ref[s0:s0 + CHK, :]
        dirs_c = b_ref[s0:s0 + CHK, 6:7]
        dd = jnp.abs(dirs_c - colT(ref_row, s0))
        dd = jnp.where(dd > PI_C, 2.0 * PI_C - dd, dd)
        mgt_c = (dd > PI_C / 2.0).astype(f32)
        w = jnp.where(same_s, s_c, 0.0)
        sg_row = sg_row + jnp.sum(w * mgt_c, axis=0, keepdims=True)
        sle_row = sle_row + jnp.sum(w * (1.0 - mgt_c), axis=0, keepdims=True)
        ssum_row = ssum_row + jnp.sum(w, axis=0, keepdims=True)

    a4 = jax.lax.broadcasted_iota(jnp.int32, (4, 1), 0)
    sub_nf = jax.lax.broadcasted_iota(jnp.int32, (NB, 1), 0).astype(f32)
    out9 = jnp.zeros((NB, 9), f32)
    for s0 in range(0, NB, CHK):
        seg_slice = jax.lax.slice(segf_row, (0, s0), (1, s0 + CHK))
        mm_s = (sub_nf == seg_slice).astype(f32)
        s_c = sc_ref[s0:s0 + CHK, :]
        dirs_c = b_ref[s0:s0 + CHK, 6:7]
        dd = jnp.abs(dirs_c - colT(ref_row, s0))
        dd = jnp.where(dd > PI_C, 2.0 * PI_C - dd, dd)
        mgt_c = (dd > PI_C / 2.0).astype(f32)
        addf = jnp.where(colT(sg_row, s0) <= colT(sle_row, s0),
                         mgt_c, 1.0 - mgt_c)
        dirs2 = _limit_period(dirs_c + addf * PI_C)
        ssum_c = colT(ssum_row, s0)
        snorm = s_c / jnp.where(ssum_c > 0.0, ssum_c, 1.0)
        term = jnp.exp((colT(rank_row, s0) + 1.0) * jnp.log(s_c))
        sx_a, sy_a = _agent_shift(t_ref, a4 == (s0 // NPER_C))
        x2_s = jnp.concatenate(
            [(b_ref[s0:s0 + CHK, 0:1] + sx_a) * snorm,
             (b_ref[s0:s0 + CHK, 1:2] + sy_a) * snorm,
             b_ref[s0:s0 + CHK, 2:3] * snorm,
             b_ref[s0:s0 + CHK, 3:4] * snorm,
             b_ref[s0:s0 + CHK, 4:5] * snorm,
             b_ref[s0:s0 + CHK, 5:6] * snorm,
             jnp.sin(dirs2) * snorm, jnp.cos(dirs2) * snorm, term], axis=1)
        out9 = out9 + jnp.dot(mm_s, x2_s, preferred_element_type=f32)

    theta = jnp.arctan2(out9[:, 6:7], out9[:, 7:8])
    sf = jnp.minimum(out9[:, 8:9], 1.0)
    out_ref[:, :] = jnp.concatenate([out9[:, 0:6], theta, sf], axis=1)


def kernel(det_boxes, det_scores, translations):
    f32 = jnp.float32
    boxes = det_boxes.astype(f32).reshape(NB, 7)
    bT = boxes.T
    s = det_scores.astype(f32).reshape(NB)
    s_col = s.reshape(NB, 1)
    s_row = s.reshape(1, NB)
    t = translations.astype(f32)

    g = NB // TILE
    adj = pl.pallas_call(
        _adj_kernel,
        grid=(g, g),
        in_specs=[
            pl.BlockSpec((TILE, 7), lambda i, j: (i, 0)),
            pl.BlockSpec((7, TILE), lambda i, j: (0, j)),
            pl.BlockSpec((4, 3), lambda i, j: (0, 0)),
        ],
        out_specs=pl.BlockSpec((TILE, TILE), lambda i, j: (i, j)),
        out_shape=jax.ShapeDtypeStruct((NB, NB), f32),
        compiler_params=pltpu.CompilerParams(
            dimension_semantics=("parallel", "arbitrary")),
        interpret=_INTERPRET,
    )(boxes, bT, t)

    out = pl.pallas_call(
        _fuse_kernel,
        out_shape=jax.ShapeDtypeStruct((NB, 8), f32),
        scratch_shapes=[pltpu.VMEM((NB, NB), f32)],
        interpret=_INTERPRET,
    )(adj, boxes, bT, s_col, s_row, t)
    return out
